# baseline (speedup 1.0000x reference)
import functools

import jax
import jax.numpy as jnp
from jax import lax
from jax.experimental import pallas as pl
from jax.experimental.pallas import tpu as pltpu
from jax.experimental.pallas import tpu_sc as plsc

N = 10000
E = 320000
D = 128
L = 3
BN_EPS = 1e-5

NC = 2
NS = 16
NW = NC * NS
CHUNK = 112
NR = 3
NI = 6
CPW = 90
E_PAD = CPW * NW * CHUNK
N_ACC = 10112
RPW = N_ACC // NS

_mesh = plsc.VectorSubcoreMesh(core_axis_name="c", subcore_axis_name="s")


@functools.partial(
    pl.kernel,
    out_type=jax.ShapeDtypeStruct((NC, N_ACC, D), jnp.float32),
    mesh=_mesh,
    scratch_types=[
        [pltpu.VMEM((CHUNK,), jnp.int32) for _ in range(NI)],
        [pltpu.VMEM((CHUNK,), jnp.int32) for _ in range(NI)],
        [pltpu.VMEM((CHUNK, D), jnp.float32) for _ in range(NR)],
        pltpu.VMEM_SHARED((N_ACC, D), jnp.float32),
        [pltpu.SemaphoreType.DMA for _ in range(NI)],
        [pltpu.SemaphoreType.DMA for _ in range(NI)],
        [pltpu.SemaphoreType.DMA for _ in range(NR)],
        [pltpu.SemaphoreType.DMA for _ in range(NR)],
    ],
)
def _sc_agg(h_hbm, src_hbm, dst_hbm, out_hbm, src_c, dst_c, rows, acc_sh,
            isrc, idst, gsem, ssem):
    c = lax.axis_index("c")
    s = lax.axis_index("s")
    wid = c * NS + s

    def _idx_arm(j, i):
        pltpu.async_copy(src_hbm.at[wid, j], src_c[i], isrc[i])
        pltpu.async_copy(dst_hbm.at[wid, j], dst_c[i], idst[i])

    def _idx_wait(j, i):
        pltpu.make_async_copy(src_hbm.at[wid, j], src_c[i], isrc[i]).wait()
        pltpu.make_async_copy(dst_hbm.at[wid, j], dst_c[i], idst[i]).wait()

    def _gather_arm(i, b):
        pltpu.async_copy(h_hbm.at[src_c[i]], rows[b], gsem[b])

    def _gather_wait(i, b):
        pltpu.make_async_copy(h_hbm.at[src_c[i]], rows[b], gsem[b]).wait()

    def _scatter_arm(i, b):
        pltpu.async_copy(rows[b], acc_sh.at[dst_c[i]], ssem[b], add=True)

    def _scatter_wait(i, b):
        pltpu.make_async_copy(rows[b], acc_sh.at[dst_c[i]], ssem[b]).wait()

    def _step(j, k, arm_gather=True, arm_idx=True, first=False):
        b, i = k % NR, k
        b2, i2 = (k + 2) % NR, (k + 2) % NI
        i5 = (k + 5) % NI
        _gather_wait(i, b)
        _scatter_arm(i, b)
        if arm_gather:
            if not first:
                _scatter_wait(i5, b2)
            _idx_wait(j + 2, i2)
            _gather_arm(i2, b2)
        if arm_idx:
            _idx_arm(j + 5, i5)

    for jj in range(5):
        _idx_arm(jj, jj)
    _idx_wait(0, 0)
    _gather_arm(0, 0)
    _idx_wait(1, 1)
    _gather_arm(1, 1)

    def _z(i, carry):
        rows[2][i // (D // 16), pl.ds((i % (D // 16)) * 16, 16)] = (
            jnp.zeros((16,), jnp.float32))
        return carry
    lax.fori_loop(0, CHUNK * (D // 16), _z, 0)
    base = s * RPW
    for off in range(0, RPW, CHUNK):
        sz = min(CHUNK, RPW - off)
        pltpu.sync_copy(rows[2].at[pl.ds(0, sz)], acc_sh.at[pl.ds(base + off, sz)])
    plsc.subcore_barrier()

    _step(0, 0, first=True)
    for jj in range(1, 6):
        _step(jj, jj % NI)

    def _body(t, carry):
        j0 = t * NI
        for k in range(NI):
            _step(j0 + k, k)
        return carry
    lax.fori_loop(1, CPW // NI - 1, _body, 0)

    for jj in range(CPW - 6, CPW):
        _step(jj, jj % NI,
              arm_gather=(jj + 2 < CPW), arm_idx=(jj + 5 < CPW))
    for jj in range(CPW - 3, CPW):
        _scatter_wait(jj % NI, jj % NR)
    plsc.subcore_barrier()

    pltpu.sync_copy(acc_sh.at[pl.ds(base, RPW)], out_hbm.at[c, pl.ds(base, RPW)])


TC_BLK = 1000
TC_NB = N // TC_BLK


def _tc_body(h_ref, agg_ref, w1_ref, b1_ref, w2_ref, b2_ref, g_ref, be_ref, o_ref,
             z_sc, st_sc, *, relu_out):
    p = pl.program_id(0)
    b = pl.program_id(1)

    @pl.when(p == 0)
    def _phase0():
        z = h_ref[...] + agg_ref[0] + agg_ref[1]
        z = lax.dot(z, w1_ref[...], preferred_element_type=jnp.float32) + b1_ref[...]
        z = jnp.maximum(z, 0.0)
        z = lax.dot(z, w2_ref[...], preferred_element_type=jnp.float32) + b2_ref[...]
        z_sc[pl.ds(b * TC_BLK, TC_BLK), :] = z
        s1 = jnp.sum(z, axis=0, keepdims=True)
        s2 = jnp.sum(z * z, axis=0, keepdims=True)

        @pl.when(b == 0)
        def _():
            st_sc[0:1, :] = s1
            st_sc[1:2, :] = s2

        @pl.when(b != 0)
        def _():
            st_sc[0:1, :] = st_sc[0:1, :] + s1
            st_sc[1:2, :] = st_sc[1:2, :] + s2

    @pl.when(p == 1)
    def _phase1():
        z = z_sc[pl.ds(b * TC_BLK, TC_BLK), :]
        m = st_sc[0:1, :] * (1.0 / N)
        v = st_sc[1:2, :] * (1.0 / N) - m * m
        z = (z - m) * lax.rsqrt(v + BN_EPS) * g_ref[...] + be_ref[...]
        if relu_out:
            z = jnp.maximum(z, 0.0)
        o_ref[...] = z


def _tc_layer(relu_out):
    return pl.pallas_call(
        functools.partial(_tc_body, relu_out=relu_out),
        grid=(2, TC_NB),
        in_specs=[
            pl.BlockSpec((TC_BLK, D), lambda p, b: (jnp.where(p == 0, b, 0), 0)),
            pl.BlockSpec((NC, TC_BLK, D),
                         lambda p, b: (0, jnp.where(p == 0, b, 0), 0)),
            pl.BlockSpec((D, D), lambda p, b: (0, 0)),
            pl.BlockSpec((1, D), lambda p, b: (0, 0)),
            pl.BlockSpec((D, D), lambda p, b: (0, 0)),
            pl.BlockSpec((1, D), lambda p, b: (0, 0)),
            pl.BlockSpec((1, D), lambda p, b: (0, 0)),
            pl.BlockSpec((1, D), lambda p, b: (0, 0)),
        ],
        out_specs=pl.BlockSpec((TC_BLK, D), lambda p, b: (jnp.where(p == 1, b, 0), 0)),
        out_shape=jax.ShapeDtypeStruct((N, D), jnp.float32),
        scratch_shapes=[
            pltpu.VMEM((N, D), jnp.float32),
            pltpu.VMEM((2, D), jnp.float32),
        ],
    )


def kernel(x, edge_index, W1, b1, W2, b2, gamma, beta):
    src = edge_index[0].astype(jnp.int32)
    dst = edge_index[1].astype(jnp.int32)
    pad = E_PAD - E
    src_p = jnp.concatenate([src, jnp.arange(pad, dtype=jnp.int32) % N])
    junk = N + jnp.arange(pad, dtype=jnp.int32) % (N_ACC - N)
    dst_p = jnp.concatenate([dst, junk])
    src3 = src_p.reshape(CPW, NW, CHUNK).transpose(1, 0, 2)
    dst3 = dst_p.reshape(CPW, NW, CHUNK).transpose(1, 0, 2)

    h = x
    for i in range(L):
        agg = _sc_agg(h, src3, dst3)
        h = _tc_layer(relu_out=(i != L - 1))(
            h, agg, W1[i], b1[i].reshape(1, D), W2[i], b2[i].reshape(1, D),
            gamma[i].reshape(1, D), beta[i].reshape(1, D))
    return h


# speedup vs baseline: 1.0372x; 1.0372x over previous
import functools

import jax
import jax.numpy as jnp
from jax import lax
from jax.experimental import pallas as pl
from jax.experimental.pallas import tpu as pltpu
from jax.experimental.pallas import tpu_sc as plsc

N = 10000
E = 320000
D = 128
L = 3
BN_EPS = 1e-5

NC = 2
NS = 16
NW = NC * NS
CHUNK = 112
NR = 3
NI = 6
CPW = 90
E_PAD = CPW * NW * CHUNK
N_ACC = 10112
RPW = N_ACC // NS

_mesh = plsc.VectorSubcoreMesh(core_axis_name="c", subcore_axis_name="s")


@functools.partial(
    pl.kernel,
    out_type=jax.ShapeDtypeStruct((NC, N_ACC, D), jnp.float32),
    mesh=_mesh,
    scratch_types=[
        [pltpu.VMEM((CHUNK,), jnp.int32) for _ in range(NI)],
        [pltpu.VMEM((CHUNK,), jnp.int32) for _ in range(NI)],
        [pltpu.VMEM((CHUNK, D), jnp.float32) for _ in range(NR)],
        pltpu.VMEM_SHARED((N_ACC, D), jnp.float32),
        [pltpu.SemaphoreType.DMA for _ in range(NI)],
        [pltpu.SemaphoreType.DMA for _ in range(NI)],
        [pltpu.SemaphoreType.DMA for _ in range(NR)],
        [pltpu.SemaphoreType.DMA for _ in range(NR)],
    ],
)
def _sc_agg(h_hbm, src_hbm, dst_hbm, out_hbm, src_c, dst_c, rows, acc_sh,
            isrc, idst, gsem, ssem):
    c = lax.axis_index("c")
    s = lax.axis_index("s")
    wid = c * NS + s

    def _idx_arm(j, i):
        pltpu.async_copy(src_hbm.at[wid, j], src_c[i], isrc[i])
        pltpu.async_copy(dst_hbm.at[wid, j], dst_c[i], idst[i])

    def _idx_wait(j, i):
        pltpu.make_async_copy(src_hbm.at[wid, j], src_c[i], isrc[i]).wait()
        pltpu.make_async_copy(dst_hbm.at[wid, j], dst_c[i], idst[i]).wait()

    def _gather_arm(i, b):
        pltpu.async_copy(h_hbm.at[src_c[i]], rows[b], gsem[b])

    def _gather_wait(i, b):
        pltpu.make_async_copy(h_hbm.at[src_c[i]], rows[b], gsem[b]).wait()

    def _scatter_arm(i, b):
        pltpu.async_copy(rows[b], acc_sh.at[dst_c[i]], ssem[b], add=True)

    def _scatter_wait(i, b):
        pltpu.make_async_copy(rows[b], acc_sh.at[dst_c[i]], ssem[b]).wait()

    def _step(j, k, arm_gather=True, arm_idx=True, first=False):
        b, i = k % NR, k
        b2, i2 = (k + 2) % NR, (k + 2) % NI
        i5 = (k + 5) % NI
        _gather_wait(i, b)
        _scatter_arm(i, b)
        if arm_gather:
            if not first:
                _scatter_wait(i5, b2)
            _idx_wait(j + 2, i2)
            _gather_arm(i2, b2)
        if arm_idx:
            _idx_arm(j + 5, i5)

    for jj in range(5):
        _idx_arm(jj, jj)
    _idx_wait(0, 0)
    _gather_arm(0, 0)
    _idx_wait(1, 1)
    _gather_arm(1, 1)

    def _z(i, carry):
        rows[2][i // (D // 16), pl.ds((i % (D // 16)) * 16, 16)] = (
            jnp.zeros((16,), jnp.float32))
        return carry
    lax.fori_loop(0, CHUNK * (D // 16), _z, 0)
    base = s * RPW
    for off in range(0, RPW, CHUNK):
        sz = min(CHUNK, RPW - off)
        pltpu.sync_copy(rows[2].at[pl.ds(0, sz)], acc_sh.at[pl.ds(base + off, sz)])
    plsc.subcore_barrier()

    _step(0, 0, first=True)
    for jj in range(1, 6):
        _step(jj, jj % NI)

    def _body(t, carry):
        j0 = t * NI
        for k in range(NI):
            _step(j0 + k, k)
        return carry
    lax.fori_loop(1, CPW // NI - 1, _body, 0)

    for jj in range(CPW - 6, CPW):
        _step(jj, jj % NI,
              arm_gather=(jj + 2 < CPW), arm_idx=(jj + 5 < CPW))
    for jj in range(CPW - 3, CPW):
        _scatter_wait(jj % NI, jj % NR)
    plsc.subcore_barrier()

    pltpu.sync_copy(acc_sh.at[pl.ds(base, RPW)], out_hbm.at[c, pl.ds(base, RPW)])


def _tc_body(h_ref, agg_ref, w1_ref, b1_ref, w2_ref, b2_ref, g_ref, be_ref, o_ref,
             *, relu_out):
    z = h_ref[...] + agg_ref[0, :N, :] + agg_ref[1, :N, :]
    z = lax.dot(z, w1_ref[...], preferred_element_type=jnp.float32) + b1_ref[...]
    z = jnp.maximum(z, 0.0)
    z = lax.dot(z, w2_ref[...], preferred_element_type=jnp.float32) + b2_ref[...]
    m = jnp.mean(z, axis=0, keepdims=True)
    d = z - m
    v = jnp.mean(d * d, axis=0, keepdims=True)
    z = d * lax.rsqrt(v + BN_EPS) * g_ref[...] + be_ref[...]
    if relu_out:
        z = jnp.maximum(z, 0.0)
    o_ref[...] = z


def _tc_layer(relu_out):
    return pl.pallas_call(
        functools.partial(_tc_body, relu_out=relu_out),
        out_shape=jax.ShapeDtypeStruct((N, D), jnp.float32),
    )


def kernel(x, edge_index, W1, b1, W2, b2, gamma, beta):
    src = edge_index[0].astype(jnp.int32)
    dst = edge_index[1].astype(jnp.int32)
    pad = E_PAD - E
    src_p = jnp.concatenate([src, jnp.arange(pad, dtype=jnp.int32) % N])
    junk = N + jnp.arange(pad, dtype=jnp.int32) % (N_ACC - N)
    dst_p = jnp.concatenate([dst, junk])
    src3 = src_p.reshape(CPW, NW, CHUNK).transpose(1, 0, 2)
    dst3 = dst_p.reshape(CPW, NW, CHUNK).transpose(1, 0, 2)

    h = x
    for i in range(L):
        agg = _sc_agg(h, src3, dst3)
        h = _tc_layer(relu_out=(i != L - 1))(
            h, agg, W1[i], b1[i].reshape(1, D), W2[i], b2[i].reshape(1, D),
            gamma[i].reshape(1, D), beta[i].reshape(1, D))
    return h
